# baseline (speedup 1.0000x reference)
import functools

import numpy as np
import jax
import jax.numpy as jnp
from jax import lax
from jax.experimental import pallas as pl
from jax.experimental.pallas import tpu as pltpu
from jax.experimental.pallas import tpu_sc as plsc

_FEAT = 2
_NL = 16
_MAX_RES, _MIN_RES = 2048, 16
_MAX_ENTRY = 2 ** 19
_MASK = _MAX_ENTRY - 1
_factor = np.exp((np.log(_MAX_RES) - np.log(_MIN_RES)) / (_NL - 1))
_RES = [float(np.floor(_MIN_RES * _factor ** i)) for i in range(_NL)]
_SIZES = [int(min(r ** 3, _MAX_ENTRY)) for r in _RES]
_ROW_OFF = [int(v) for v in np.cumsum([0] + _SIZES)[:-1]]
_TOTAL_ROWS = int(np.sum(_SIZES))
_PRIMES = (3367900313, 2654435761, 805459861)
_P32 = [int(p - 2 ** 32 if p >= 2 ** 31 else p) for p in _PRIMES]
_CORNERS = [(0, 0, 0), (0, 1, 0), (0, 0, 1), (0, 1, 1),
            (1, 0, 0), (1, 0, 1), (1, 1, 0), (1, 1, 1)]

_NC, _NS, _L = 2, 16, 16
_NW = _NC * _NS
_C = 256
_G = _C // _L
_B = 8 * _C
_B2 = 2 * _B
_NI = _B2 // 128

_pf = np.zeros((_NL, 2, _L), np.float32)
_pi = np.zeros((_NL, 4, _L), np.int32)
for _l in range(_NL):
    _r = _RES[_l]
    _pf[_l, 0] = np.float32(_r - 1)
    _pf[_l, 1] = np.float32(_r - 1.0001)
    _ri = int(_r)
    _pi[_l, 0] = _ri
    _pi[_l, 1] = _ri * _ri
    _pi[_l, 2] = 1 if _SIZES[_l] == _MAX_ENTRY else 0
    _pi[_l, 3] = _ROW_OFF[_l]
_PF = _pf.reshape(-1)
_PI = _pi.reshape(-1)


def _grid_body(npt, n_pts, xf_hbm, tab_hbm, pf_hbm, pi_hbm, out_hbm,
               x_v, pf_v, pi_v, idx_v, w_v, rows_v, out_v, sem):
    wid = lax.axis_index("s") * _NC + lax.axis_index("c")
    pt0 = wid * npt
    pltpu.sync_copy(xf_hbm.at[pl.ds(3 * pt0, 3 * npt)], x_v)
    pltpu.sync_copy(pf_hbm, pf_v)
    pltpu.sync_copy(pi_hbm, pi_v)
    iota = lax.iota(jnp.int32, _L)
    iota3 = iota * 3
    n_steps = (npt // _C) * _NL

    def p_phase(t):
        chunk = t >> 4
        lvl = t & 15
        slot = t & 1
        pbase = chunk * _C
        fo = lvl * (2 * _L)
        res_m1 = pf_v[pl.ds(fo, _L)]
        clip_hi = pf_v[pl.ds(fo + _L, _L)]
        io = lvl * (4 * _L)
        r_v = pi_v[pl.ds(io, _L)]
        r2_v = pi_v[pl.ds(io + _L, _L)]
        hashed_v = pi_v[pl.ds(io + 2 * _L, _L)]
        off_v = pi_v[pl.ds(io + 3 * _L, _L)]
        hmask = hashed_v > 0
        slot_b = jnp.full((_L,), slot, jnp.int32)

        def g_body(g, carry):
            p0 = pbase + g * _L
            xb = 3 * p0 + iota3
            x0 = plsc.load_gather(x_v, [xb])
            x1 = plsc.load_gather(x_v, [xb + 1])
            x2 = plsc.load_gather(x_v, [xb + 2])
            c0 = jnp.minimum(jnp.maximum(x0 * res_m1, 0.0), clip_hi)
            c1 = jnp.minimum(jnp.maximum(x1 * res_m1, 0.0), clip_hi)
            c2 = jnp.minimum(jnp.maximum(x2 * res_m1, 0.0), clip_hi)
            i0 = c0.astype(jnp.int32)
            i1 = c1.astype(jnp.int32)
            i2 = c2.astype(jnp.int32)
            d0 = c0 - i0.astype(jnp.float32)
            d1 = c1 - i1.astype(jnp.float32)
            d2 = c2 - i2.astype(jnp.float32)
            a0 = i0 * _P32[0]; a0b = a0 + _P32[0]
            a1 = i1 * _P32[1]; a1b = a1 + _P32[1]
            a2 = i2 * _P32[2]; a2b = a2 + _P32[2]
            b1 = i1 * r_v; b1b = b1 + r_v
            b2 = i2 * r2_v; b2b = b2 + r2_v
            i0p = i0 + 1
            mx = 1.0 - d0; my = 1.0 - d1; mz = 1.0 - d2
            wxy = (mx * my, d0 * my, mx * d1, d0 * d1)
            for j, (ox, oy, oz) in enumerate(_CORNERS):
                h = ((a0b if ox else a0) ^ (a1b if oy else a1) ^
                     (a2b if oz else a2)) & _MASK
                didx = ((i0p if ox else i0) + (b1b if oy else b1) +
                        (b2b if oz else b2))
                idx2 = (jnp.where(hmask, h, didx) + off_v) * 2
                fpos = j * _C + g * _L
                zer = jnp.zeros((_L,), jnp.int32)
                pos2 = 2 * fpos + 2 * iota
                plsc.store_scatter(idx_v, [slot_b, zer, pos2], idx2)
                plsc.store_scatter(idx_v, [slot_b, zer, pos2 + 1], idx2 + 1)
                wj = wxy[(1 if j & 1 else 0) + (2 if j & 2 else 0)]
                wj = wj * (d2 if j & 4 else mz)
                w_v[pl.ds(slot * _B + fpos, _L)] = wj
            return carry

        lax.fori_loop(jnp.int32(0), jnp.int32(_G), g_body, jnp.int32(0), unroll=False)

        pltpu.async_copy(tab_hbm.at[idx_v.at[slot, jnp.int32(0)]],
                         rows_v.at[slot, jnp.int32(0)], sem.at[slot])

    def a_phase(tp):
        chunkp = tp >> 4
        lvlp = tp & 15
        slotp = tp & 1
        cslot = chunkp & 1
        pltpu.make_async_copy(tab_hbm.at[idx_v.at[slotp, jnp.int32(0)]],
                              rows_v.at[slotp, jnp.int32(0)], sem.at[slotp]).wait()
        colbase = lvlp * 2

        def g_body(g, carry):
            obase = cslot * (_C * 32) + g * (_L * 32) + colbase
            acc0 = jnp.zeros((_L,), jnp.float32)
            acc1 = jnp.zeros((_L,), jnp.float32)
            for j in range(8):
                fpos = j * _C + g * _L
                wv = w_v[pl.ds(slotp * _B + fpos, _L)]
                slot_bb = jnp.full((_L,), slotp, jnp.int32)
                zer = jnp.zeros((_L,), jnp.int32)
                rvec2 = 2 * fpos + 2 * iota
                v0 = plsc.load_gather(rows_v, [slot_bb, zer, rvec2])
                v1 = plsc.load_gather(rows_v, [slot_bb, zer, rvec2 + 1])
                acc0 = acc0 + wv * v0
                acc1 = acc1 + wv * v1
            opos = obase + iota * 32
            plsc.store_scatter(out_v, [opos], acc0)
            plsc.store_scatter(out_v, [opos + 1], acc1)
            return carry

        lax.fori_loop(jnp.int32(0), jnp.int32(_G), g_body, jnp.int32(0), unroll=False)

        @pl.when(lvlp == jnp.int32(15))
        def _():
            gb = (pt0 + chunkp * _C) * 32
            pltpu.sync_copy(out_v.at[pl.ds(cslot * (_C * 32), _C * 32)],
                            out_hbm.at[pl.ds(gb, _C * 32)])

    def step(t, carry):
        @pl.when(t < jnp.int32(n_steps))
        def _():
            p_phase(t)

        @pl.when(t > jnp.int32(0))
        def _():
            a_phase(t - 1)

        return carry

    lax.fori_loop(jnp.int32(0), jnp.int32(n_steps + 1), step, jnp.int32(0), unroll=False)


_CHUNKW = 16384
_TOTAL_WORDS = 2 * _TOTAL_ROWS


def _stage_body(*refs):
    tabs = refs[:_NL]
    tab_hbm = refs[_NL]
    buf = refs[_NL + 1]
    wid = lax.axis_index("s") * _NC + lax.axis_index("c")
    half_sel = wid & 1
    for t in range(_NL):
        @pl.when((wid >> 1) == jnp.int32(t))
        def _(t=t):
            half = _SIZES[t]
            src0 = half_sel * half
            dst0 = 2 * _ROW_OFF[t] + src0
            nb, tail = divmod(half, _CHUNKW)

            def k_body(k, carry):
                o = k * _CHUNKW
                pltpu.sync_copy(tabs[t].at[pl.ds(src0 + o, _CHUNKW)], buf)
                pltpu.sync_copy(buf, tab_hbm.at[pl.ds(dst0 + o, _CHUNKW)])
                return carry

            lax.fori_loop(jnp.int32(0), jnp.int32(nb), k_body, jnp.int32(0),
                          unroll=False)
            if tail:
                o = nb * _CHUNKW
                pltpu.sync_copy(tabs[t].at[pl.ds(src0 + o, tail)],
                                buf.at[pl.ds(0, tail)])
                pltpu.sync_copy(buf.at[pl.ds(0, tail)],
                                tab_hbm.at[pl.ds(dst0 + o, tail)])


def kernel(x, tables):
    n_pts = x.shape[0]
    assert n_pts % (_NW * _C) == 0
    npt = n_pts // _NW
    xf = x.reshape(-1)
    flats = [t.reshape(-1) for t in tables]
    mesh = plsc.VectorSubcoreMesh(core_axis_name="c", subcore_axis_name="s")
    stage = pl.kernel(
        _stage_body,
        out_type=jax.ShapeDtypeStruct((_TOTAL_WORDS,), jnp.float32),
        mesh=mesh,
        scratch_types=[pltpu.VMEM((_CHUNKW,), jnp.float32)],
        compiler_params=pltpu.CompilerParams(needs_layout_passes=False),
    )
    tab = stage(*flats)
    kfn = pl.kernel(
        functools.partial(_grid_body, npt, n_pts),
        out_type=jax.ShapeDtypeStruct((n_pts * 2 * _NL,), jnp.float32),
        mesh=mesh,
        scratch_types=[
            pltpu.VMEM((3 * npt,), jnp.float32),
            pltpu.VMEM((_NL * 2 * _L,), jnp.float32),
            pltpu.VMEM((_NL * 4 * _L,), jnp.int32),
            pltpu.VMEM((2, 1, _B2), jnp.int32),
            pltpu.VMEM((2 * _B,), jnp.float32),
            pltpu.VMEM((2, 1, _B2), jnp.float32),
            pltpu.VMEM((2 * _C * 32,), jnp.float32),
            pltpu.SemaphoreType.DMA((2,)),
        ],
        compiler_params=pltpu.CompilerParams(needs_layout_passes=False),
    )
    out = kfn(xf, tab, jnp.asarray(_PF), jnp.asarray(_PI))
    return out.reshape(n_pts, 2 * _NL)


# speedup vs baseline: 1.0081x; 1.0081x over previous
import functools

import numpy as np
import jax
import jax.numpy as jnp
from jax import lax
from jax.experimental import pallas as pl
from jax.experimental.pallas import tpu as pltpu
from jax.experimental.pallas import tpu_sc as plsc

_FEAT = 2
_NL = 16
_MAX_RES, _MIN_RES = 2048, 16
_MAX_ENTRY = 2 ** 19
_MASK = _MAX_ENTRY - 1
_factor = np.exp((np.log(_MAX_RES) - np.log(_MIN_RES)) / (_NL - 1))
_RES = [float(np.floor(_MIN_RES * _factor ** i)) for i in range(_NL)]
_SIZES = [int(min(r ** 3, _MAX_ENTRY)) for r in _RES]
_ROW_OFF = [int(v) for v in np.cumsum([0] + _SIZES)[:-1]]
_TOTAL_ROWS = int(np.sum(_SIZES))
_PRIMES = (3367900313, 2654435761, 805459861)
_P32 = [int(p - 2 ** 32 if p >= 2 ** 31 else p) for p in _PRIMES]
_CORNERS = [(0, 0, 0), (0, 1, 0), (0, 0, 1), (0, 1, 1),
            (1, 0, 0), (1, 0, 1), (1, 1, 0), (1, 1, 1)]

_NC, _NS, _L = 2, 16, 16
_NW = _NC * _NS
_C = 256
_G = _C // _L
_B = 8 * _C
_B2 = 2 * _B
_NI = _B2 // 128

_pf = np.zeros((_NL, 2, _L), np.float32)
_pi = np.zeros((_NL, 4, _L), np.int32)
for _l in range(_NL):
    _r = _RES[_l]
    _pf[_l, 0] = np.float32(_r - 1)
    _pf[_l, 1] = np.float32(_r - 1.0001)
    _ri = int(_r)
    _pi[_l, 0] = _ri
    _pi[_l, 1] = _ri * _ri
    _pi[_l, 2] = 1 if _SIZES[_l] == _MAX_ENTRY else 0
    _pi[_l, 3] = _ROW_OFF[_l]
_PF = _pf.reshape(-1)
_PI = _pi.reshape(-1)


_CHUNKW = 16384
_TOTAL_WORDS = 2 * _TOTAL_ROWS


def _grid_body(npt, n_pts, *refs):
    xf_hbm = refs[0]
    tab_in = refs[1:1 + _NL]
    pf_hbm, pi_hbm, out_hbm = refs[1 + _NL:4 + _NL]
    (x_v, pf_v, pi_v, idx_v, w_v, rows_v, out_v, stage_v, tab_hbm,
     sem) = refs[4 + _NL:]
    s_id = lax.axis_index("s")
    wid = s_id * _NC + lax.axis_index("c")
    pt0 = wid * npt
    for t in range(_NL):
        @pl.when(s_id == jnp.int32(t))
        def _(t=t):
            w_total = 2 * _SIZES[t]
            dst0 = 2 * _ROW_OFF[t]
            nb, tail = divmod(w_total, _CHUNKW)

            def k_body(k, carry):
                o = k * _CHUNKW
                pltpu.sync_copy(tab_in[t].at[pl.ds(o, _CHUNKW)], stage_v)
                pltpu.sync_copy(stage_v, tab_hbm.at[pl.ds(dst0 + o, _CHUNKW)])
                return carry

            lax.fori_loop(jnp.int32(0), jnp.int32(nb), k_body, jnp.int32(0),
                          unroll=False)
            if tail:
                o = nb * _CHUNKW
                pltpu.sync_copy(tab_in[t].at[pl.ds(o, tail)],
                                stage_v.at[pl.ds(0, tail)])
                pltpu.sync_copy(stage_v.at[pl.ds(0, tail)],
                                tab_hbm.at[pl.ds(dst0 + o, tail)])
    plsc.subcore_barrier()
    pltpu.sync_copy(xf_hbm.at[pl.ds(3 * pt0, 3 * npt)], x_v)
    pltpu.sync_copy(pf_hbm, pf_v)
    pltpu.sync_copy(pi_hbm, pi_v)
    iota = lax.iota(jnp.int32, _L)
    iota3 = iota * 3
    n_steps = (npt // _C) * _NL

    def p_phase(t):
        chunk = t >> 4
        lvl = t & 15
        slot = t & 1
        pbase = chunk * _C
        fo = lvl * (2 * _L)
        res_m1 = pf_v[pl.ds(fo, _L)]
        clip_hi = pf_v[pl.ds(fo + _L, _L)]
        io = lvl * (4 * _L)
        r_v = pi_v[pl.ds(io, _L)]
        r2_v = pi_v[pl.ds(io + _L, _L)]
        hashed_v = pi_v[pl.ds(io + 2 * _L, _L)]
        off_v = pi_v[pl.ds(io + 3 * _L, _L)]
        hmask = hashed_v > 0
        slot_b = jnp.full((_L,), slot, jnp.int32)

        def g_body(g, carry):
            p0 = pbase + g * _L
            xb = 3 * p0 + iota3
            x0 = plsc.load_gather(x_v, [xb])
            x1 = plsc.load_gather(x_v, [xb + 1])
            x2 = plsc.load_gather(x_v, [xb + 2])
            c0 = jnp.minimum(jnp.maximum(x0 * res_m1, 0.0), clip_hi)
            c1 = jnp.minimum(jnp.maximum(x1 * res_m1, 0.0), clip_hi)
            c2 = jnp.minimum(jnp.maximum(x2 * res_m1, 0.0), clip_hi)
            i0 = c0.astype(jnp.int32)
            i1 = c1.astype(jnp.int32)
            i2 = c2.astype(jnp.int32)
            d0 = c0 - i0.astype(jnp.float32)
            d1 = c1 - i1.astype(jnp.float32)
            d2 = c2 - i2.astype(jnp.float32)
            a0 = i0 * _P32[0]; a0b = a0 + _P32[0]
            a1 = i1 * _P32[1]; a1b = a1 + _P32[1]
            a2 = i2 * _P32[2]; a2b = a2 + _P32[2]
            b1 = i1 * r_v; b1b = b1 + r_v
            b2 = i2 * r2_v; b2b = b2 + r2_v
            i0p = i0 + 1
            mx = 1.0 - d0; my = 1.0 - d1; mz = 1.0 - d2
            wxy = (mx * my, d0 * my, mx * d1, d0 * d1)
            for j, (ox, oy, oz) in enumerate(_CORNERS):
                h = ((a0b if ox else a0) ^ (a1b if oy else a1) ^
                     (a2b if oz else a2)) & _MASK
                didx = ((i0p if ox else i0) + (b1b if oy else b1) +
                        (b2b if oz else b2))
                idx2 = (jnp.where(hmask, h, didx) + off_v) * 2
                fpos = j * _C + g * _L
                zer = jnp.zeros((_L,), jnp.int32)
                pos2 = 2 * fpos + 2 * iota
                plsc.store_scatter(idx_v, [slot_b, zer, pos2], idx2)
                plsc.store_scatter(idx_v, [slot_b, zer, pos2 + 1], idx2 + 1)
                wj = wxy[(1 if j & 1 else 0) + (2 if j & 2 else 0)]
                wj = wj * (d2 if j & 4 else mz)
                w_v[pl.ds(slot * _B + fpos, _L)] = wj
            return carry

        lax.fori_loop(jnp.int32(0), jnp.int32(_G), g_body, jnp.int32(0), unroll=False)

        pltpu.async_copy(tab_hbm.at[idx_v.at[slot, jnp.int32(0)]],
                         rows_v.at[slot, jnp.int32(0)], sem.at[slot])

    def a_phase(tp):
        chunkp = tp >> 4
        lvlp = tp & 15
        slotp = tp & 1
        cslot = chunkp & 1
        pltpu.make_async_copy(tab_hbm.at[idx_v.at[slotp, jnp.int32(0)]],
                              rows_v.at[slotp, jnp.int32(0)], sem.at[slotp]).wait()
        colbase = lvlp * 2

        def g_body(g, carry):
            obase = cslot * (_C * 32) + g * (_L * 32) + colbase
            acc0 = jnp.zeros((_L,), jnp.float32)
            acc1 = jnp.zeros((_L,), jnp.float32)
            for j in range(8):
                fpos = j * _C + g * _L
                wv = w_v[pl.ds(slotp * _B + fpos, _L)]
                slot_bb = jnp.full((_L,), slotp, jnp.int32)
                zer = jnp.zeros((_L,), jnp.int32)
                rvec2 = 2 * fpos + 2 * iota
                v0 = plsc.load_gather(rows_v, [slot_bb, zer, rvec2])
                v1 = plsc.load_gather(rows_v, [slot_bb, zer, rvec2 + 1])
                acc0 = acc0 + wv * v0
                acc1 = acc1 + wv * v1
            opos = obase + iota * 32
            plsc.store_scatter(out_v, [opos], acc0)
            plsc.store_scatter(out_v, [opos + 1], acc1)
            return carry

        lax.fori_loop(jnp.int32(0), jnp.int32(_G), g_body, jnp.int32(0), unroll=False)

        @pl.when(lvlp == jnp.int32(15))
        def _():
            gb = (pt0 + chunkp * _C) * 32
            pltpu.sync_copy(out_v.at[pl.ds(cslot * (_C * 32), _C * 32)],
                            out_hbm.at[pl.ds(gb, _C * 32)])

    def step(t, carry):
        @pl.when(t < jnp.int32(n_steps))
        def _():
            p_phase(t)

        @pl.when(t > jnp.int32(0))
        def _():
            a_phase(t - 1)

        return carry

    lax.fori_loop(jnp.int32(0), jnp.int32(n_steps + 1), step, jnp.int32(0), unroll=False)


_CHUNKW = 16384
_TOTAL_WORDS = 2 * _TOTAL_ROWS


def _stage_body(*refs):
    tabs = refs[:_NL]
    tab_hbm = refs[_NL]
    buf = refs[_NL + 1]
    wid = lax.axis_index("s") * _NC + lax.axis_index("c")
    half_sel = wid & 1
    for t in range(_NL):
        @pl.when((wid >> 1) == jnp.int32(t))
        def _(t=t):
            half = _SIZES[t]
            src0 = half_sel * half
            dst0 = 2 * _ROW_OFF[t] + src0
            nb, tail = divmod(half, _CHUNKW)

            def k_body(k, carry):
                o = k * _CHUNKW
                pltpu.sync_copy(tabs[t].at[pl.ds(src0 + o, _CHUNKW)], buf)
                pltpu.sync_copy(buf, tab_hbm.at[pl.ds(dst0 + o, _CHUNKW)])
                return carry

            lax.fori_loop(jnp.int32(0), jnp.int32(nb), k_body, jnp.int32(0),
                          unroll=False)
            if tail:
                o = nb * _CHUNKW
                pltpu.sync_copy(tabs[t].at[pl.ds(src0 + o, tail)],
                                buf.at[pl.ds(0, tail)])
                pltpu.sync_copy(buf.at[pl.ds(0, tail)],
                                tab_hbm.at[pl.ds(dst0 + o, tail)])


def kernel(x, tables):
    n_pts = x.shape[0]
    assert n_pts % (_NW * _C) == 0
    npt = n_pts // _NW
    xf = x.reshape(-1)
    flats = [t.reshape(-1) for t in tables]
    mesh = plsc.VectorSubcoreMesh(core_axis_name="c", subcore_axis_name="s")
    kfn = pl.kernel(
        functools.partial(_grid_body, npt, n_pts),
        out_type=jax.ShapeDtypeStruct((n_pts * 2 * _NL,), jnp.float32),
        mesh=mesh,
        scratch_types=[
            pltpu.VMEM((3 * npt,), jnp.float32),
            pltpu.VMEM((_NL * 2 * _L,), jnp.float32),
            pltpu.VMEM((_NL * 4 * _L,), jnp.int32),
            pltpu.VMEM((2, 1, _B2), jnp.int32),
            pltpu.VMEM((2 * _B,), jnp.float32),
            pltpu.VMEM((2, 1, _B2), jnp.float32),
            pltpu.VMEM((2 * _C * 32,), jnp.float32),
            pltpu.VMEM((_CHUNKW,), jnp.float32),
            pltpu.HBM((_TOTAL_WORDS,), jnp.float32),
            pltpu.SemaphoreType.DMA((2,)),
        ],
        compiler_params=pltpu.CompilerParams(needs_layout_passes=False),
    )
    out = kfn(xf, *flats, jnp.asarray(_PF), jnp.asarray(_PI))
    return out.reshape(n_pts, 2 * _NL)


# speedup vs baseline: 1.0801x; 1.0715x over previous
import functools

import numpy as np
import jax
import jax.numpy as jnp
from jax import lax
from jax.experimental import pallas as pl
from jax.experimental.pallas import tpu as pltpu
from jax.experimental.pallas import tpu_sc as plsc

_FEAT = 2
_NL = 16
_MAX_RES, _MIN_RES = 2048, 16
_MAX_ENTRY = 2 ** 19
_MASK = _MAX_ENTRY - 1
_factor = np.exp((np.log(_MAX_RES) - np.log(_MIN_RES)) / (_NL - 1))
_RES = [float(np.floor(_MIN_RES * _factor ** i)) for i in range(_NL)]
_SIZES = [int(min(r ** 3, _MAX_ENTRY)) for r in _RES]
_ROW_OFF = [int(v) for v in np.cumsum([0] + _SIZES)[:-1]]
_TOTAL_ROWS = int(np.sum(_SIZES))
_PRIMES = (3367900313, 2654435761, 805459861)
_P32 = [int(p - 2 ** 32 if p >= 2 ** 31 else p) for p in _PRIMES]
_CORNERS = [(0, 0, 0), (0, 1, 0), (0, 0, 1), (0, 1, 1),
            (1, 0, 0), (1, 0, 1), (1, 1, 0), (1, 1, 1)]

_NC, _NS, _L = 2, 16, 16
_NW = _NC * _NS
_C = 256
_G = _C // _L
_B = 8 * _C
_B2 = 2 * _B
_NI = _B2 // 128

_pf = np.zeros((_NL, 2, _L), np.float32)
_pi = np.zeros((_NL, 4, _L), np.int32)
for _l in range(_NL):
    _r = _RES[_l]
    _pf[_l, 0] = np.float32(_r - 1)
    _pf[_l, 1] = np.float32(_r - 1.0001)
    _ri = int(_r)
    _pi[_l, 0] = _ri
    _pi[_l, 1] = _ri * _ri
    _pi[_l, 2] = 1 if _SIZES[_l] == _MAX_ENTRY else 0
    _pi[_l, 3] = _ROW_OFF[_l]
_PF = _pf.reshape(-1)
_PI = _pi.reshape(-1)


_CHUNKW = 16384
_CHW2 = _CHUNKW // 2
_NSTREAM = _NL - 2
_TOTAL_WORDS = 2 * _TOTAL_ROWS


def _grid_body(npt, n_pts, *refs):
    xf_hbm = refs[0]
    tab_in = refs[1:1 + _NL]
    pf_hbm, pi_hbm, out_hbm = refs[1 + _NL:4 + _NL]
    (x_v, pf_v, pi_v, idx_v, w_v, rows_v, out_v, stage_v, cache_v, tab_hbm,
     sem) = refs[4 + _NL:]
    s_id = lax.axis_index("s")
    wid = s_id * _NC + lax.axis_index("c")
    pt0 = wid * npt
    for t in range(_NL):
        @pl.when(s_id == jnp.int32(t))
        def _(t=t):
            w_total = 2 * _SIZES[t]
            dst0 = 2 * _ROW_OFF[t]
            nb, tail = divmod(w_total, _CHW2)

            def k_body(k, carry):
                o = k * _CHW2
                pltpu.sync_copy(tab_in[t].at[pl.ds(o, _CHW2)], stage_v)
                pltpu.sync_copy(stage_v, tab_hbm.at[pl.ds(dst0 + o, _CHW2)])
                return carry

            lax.fori_loop(jnp.int32(0), jnp.int32(nb), k_body, jnp.int32(0),
                          unroll=False)
            if tail:
                o = nb * _CHW2
                pltpu.sync_copy(tab_in[t].at[pl.ds(o, tail)],
                                stage_v.at[pl.ds(0, tail)])
                pltpu.sync_copy(stage_v.at[pl.ds(0, tail)],
                                tab_hbm.at[pl.ds(dst0 + o, tail)])
    plsc.subcore_barrier()
    pltpu.sync_copy(tab_in[0], cache_v.at[pl.ds(0, 2 * _SIZES[0])])
    pltpu.sync_copy(tab_in[1],
                    cache_v.at[pl.ds(2 * _SIZES[0], 2 * _SIZES[1])])
    pltpu.sync_copy(xf_hbm.at[pl.ds(3 * pt0, 3 * npt)], x_v)
    pltpu.sync_copy(pf_hbm, pf_v)
    pltpu.sync_copy(pi_hbm, pi_v)
    iota = lax.iota(jnp.int32, _L)
    iota3 = iota * 3
    n_steps = (npt // _C) * _NSTREAM

    def cached_block(cl, pbase, cslot):
        res = _RES[cl]
        res_m1 = np.float32(res - 1)
        clip_hi = np.float32(res - 1.0001)
        ri = int(res)
        ri2 = ri * ri
        coff = 0 if cl == 0 else 2 * _SIZES[0]

        def g_body(g, carry):
            p0 = pbase + g * _L
            xb = 3 * p0 + iota3
            x0 = plsc.load_gather(x_v, [xb])
            x1 = plsc.load_gather(x_v, [xb + 1])
            x2 = plsc.load_gather(x_v, [xb + 2])
            c0 = jnp.minimum(jnp.maximum(x0 * res_m1, 0.0), clip_hi)
            c1 = jnp.minimum(jnp.maximum(x1 * res_m1, 0.0), clip_hi)
            c2 = jnp.minimum(jnp.maximum(x2 * res_m1, 0.0), clip_hi)
            i0 = c0.astype(jnp.int32)
            i1 = c1.astype(jnp.int32)
            i2 = c2.astype(jnp.int32)
            d0 = c0 - i0.astype(jnp.float32)
            d1 = c1 - i1.astype(jnp.float32)
            d2 = c2 - i2.astype(jnp.float32)
            b1 = i1 * ri; b1b = b1 + ri
            b2 = i2 * ri2; b2b = b2 + ri2
            i0p = i0 + 1
            mx = 1.0 - d0; my = 1.0 - d1; mz = 1.0 - d2
            wxy = (mx * my, d0 * my, mx * d1, d0 * d1)
            acc0 = jnp.zeros((_L,), jnp.float32)
            acc1 = jnp.zeros((_L,), jnp.float32)
            for j, (ox, oy, oz) in enumerate(_CORNERS):
                didx = ((i0p if ox else i0) + (b1b if oy else b1) +
                        (b2b if oz else b2))
                cidx2 = didx * 2 + coff
                v0 = plsc.load_gather(cache_v, [cidx2])
                v1 = plsc.load_gather(cache_v, [cidx2 + 1])
                wj = wxy[(1 if j & 1 else 0) + (2 if j & 2 else 0)]
                wj = wj * (d2 if j & 4 else mz)
                acc0 = acc0 + wj * v0
                acc1 = acc1 + wj * v1
            opos = cslot * (_C * 32) + g * (_L * 32) + 2 * cl + iota * 32
            plsc.store_scatter(out_v, [opos], acc0)
            plsc.store_scatter(out_v, [opos + 1], acc1)
            return carry

        lax.fori_loop(jnp.int32(0), jnp.int32(_G), g_body, jnp.int32(0),
                      unroll=False)

    def p_phase(t):
        chunk = t // jnp.int32(_NSTREAM)
        r = t - chunk * _NSTREAM
        lvl = r + 2
        slot = t & 1
        pbase = chunk * _C

        @pl.when(r == jnp.int32(0))
        def _():
            cached_block(0, pbase, chunk & 1)

        @pl.when(r == jnp.int32(1))
        def _():
            cached_block(1, pbase, chunk & 1)

        fo = lvl * (2 * _L)
        res_m1 = pf_v[pl.ds(fo, _L)]
        clip_hi = pf_v[pl.ds(fo + _L, _L)]
        io = lvl * (4 * _L)
        r_v = pi_v[pl.ds(io, _L)]
        r2_v = pi_v[pl.ds(io + _L, _L)]
        hashed_v = pi_v[pl.ds(io + 2 * _L, _L)]
        off_v = pi_v[pl.ds(io + 3 * _L, _L)]
        hmask = hashed_v > 0
        slot_b = jnp.full((_L,), slot, jnp.int32)

        def g_body(g, carry):
            p0 = pbase + g * _L
            xb = 3 * p0 + iota3
            x0 = plsc.load_gather(x_v, [xb])
            x1 = plsc.load_gather(x_v, [xb + 1])
            x2 = plsc.load_gather(x_v, [xb + 2])
            c0 = jnp.minimum(jnp.maximum(x0 * res_m1, 0.0), clip_hi)
            c1 = jnp.minimum(jnp.maximum(x1 * res_m1, 0.0), clip_hi)
            c2 = jnp.minimum(jnp.maximum(x2 * res_m1, 0.0), clip_hi)
            i0 = c0.astype(jnp.int32)
            i1 = c1.astype(jnp.int32)
            i2 = c2.astype(jnp.int32)
            d0 = c0 - i0.astype(jnp.float32)
            d1 = c1 - i1.astype(jnp.float32)
            d2 = c2 - i2.astype(jnp.float32)
            a0 = i0 * _P32[0]; a0b = a0 + _P32[0]
            a1 = i1 * _P32[1]; a1b = a1 + _P32[1]
            a2 = i2 * _P32[2]; a2b = a2 + _P32[2]
            b1 = i1 * r_v; b1b = b1 + r_v
            b2 = i2 * r2_v; b2b = b2 + r2_v
            i0p = i0 + 1
            mx = 1.0 - d0; my = 1.0 - d1; mz = 1.0 - d2
            wxy = (mx * my, d0 * my, mx * d1, d0 * d1)
            for j, (ox, oy, oz) in enumerate(_CORNERS):
                h = ((a0b if ox else a0) ^ (a1b if oy else a1) ^
                     (a2b if oz else a2)) & _MASK
                didx = ((i0p if ox else i0) + (b1b if oy else b1) +
                        (b2b if oz else b2))
                idx2 = (jnp.where(hmask, h, didx) + off_v) * 2
                fpos = j * _C + g * _L
                zer = jnp.zeros((_L,), jnp.int32)
                pos2 = 2 * fpos + 2 * iota
                plsc.store_scatter(idx_v, [slot_b, zer, pos2], idx2)
                plsc.store_scatter(idx_v, [slot_b, zer, pos2 + 1], idx2 + 1)
                wj = wxy[(1 if j & 1 else 0) + (2 if j & 2 else 0)]
                wj = wj * (d2 if j & 4 else mz)
                w_v[pl.ds(slot * _B + fpos, _L)] = wj
            return carry

        lax.fori_loop(jnp.int32(0), jnp.int32(_G), g_body, jnp.int32(0), unroll=False)

        pltpu.async_copy(tab_hbm.at[idx_v.at[slot, jnp.int32(0)]],
                         rows_v.at[slot, jnp.int32(0)], sem.at[slot])

    def a_phase(tp):
        chunkp = tp // jnp.int32(_NSTREAM)
        rp = tp - chunkp * _NSTREAM
        lvlp = rp + 2
        slotp = tp & 1
        cslot = chunkp & 1
        pltpu.make_async_copy(tab_hbm.at[idx_v.at[slotp, jnp.int32(0)]],
                              rows_v.at[slotp, jnp.int32(0)], sem.at[slotp]).wait()
        colbase = lvlp * 2

        def g_body(g, carry):
            obase = cslot * (_C * 32) + g * (_L * 32) + colbase
            acc0 = jnp.zeros((_L,), jnp.float32)
            acc1 = jnp.zeros((_L,), jnp.float32)
            for j in range(8):
                fpos = j * _C + g * _L
                wv = w_v[pl.ds(slotp * _B + fpos, _L)]
                slot_bb = jnp.full((_L,), slotp, jnp.int32)
                zer = jnp.zeros((_L,), jnp.int32)
                rvec2 = 2 * fpos + 2 * iota
                v0 = plsc.load_gather(rows_v, [slot_bb, zer, rvec2])
                v1 = plsc.load_gather(rows_v, [slot_bb, zer, rvec2 + 1])
                acc0 = acc0 + wv * v0
                acc1 = acc1 + wv * v1
            opos = obase + iota * 32
            plsc.store_scatter(out_v, [opos], acc0)
            plsc.store_scatter(out_v, [opos + 1], acc1)
            return carry

        lax.fori_loop(jnp.int32(0), jnp.int32(_G), g_body, jnp.int32(0), unroll=False)

        @pl.when(rp == jnp.int32(_NSTREAM - 1))
        def _():
            gb = (pt0 + chunkp * _C) * 32
            pltpu.sync_copy(out_v.at[pl.ds(cslot * (_C * 32), _C * 32)],
                            out_hbm.at[pl.ds(gb, _C * 32)])

    def step(t, carry):
        @pl.when(t < jnp.int32(n_steps))
        def _():
            p_phase(t)

        @pl.when(t > jnp.int32(0))
        def _():
            a_phase(t - 1)

        return carry

    lax.fori_loop(jnp.int32(0), jnp.int32(n_steps + 1), step, jnp.int32(0), unroll=False)


_CHUNKW = 16384
_CHW2 = _CHUNKW // 2
_NSTREAM = _NL - 2
_TOTAL_WORDS = 2 * _TOTAL_ROWS


def _stage_body(*refs):
    tabs = refs[:_NL]
    tab_hbm = refs[_NL]
    buf = refs[_NL + 1]
    wid = lax.axis_index("s") * _NC + lax.axis_index("c")
    half_sel = wid & 1
    for t in range(_NL):
        @pl.when((wid >> 1) == jnp.int32(t))
        def _(t=t):
            half = _SIZES[t]
            src0 = half_sel * half
            dst0 = 2 * _ROW_OFF[t] + src0
            nb, tail = divmod(half, _CHUNKW)

            def k_body(k, carry):
                o = k * _CHUNKW
                pltpu.sync_copy(tabs[t].at[pl.ds(src0 + o, _CHUNKW)], buf)
                pltpu.sync_copy(buf, tab_hbm.at[pl.ds(dst0 + o, _CHUNKW)])
                return carry

            lax.fori_loop(jnp.int32(0), jnp.int32(nb), k_body, jnp.int32(0),
                          unroll=False)
            if tail:
                o = nb * _CHW2
                pltpu.sync_copy(tabs[t].at[pl.ds(src0 + o, tail)],
                                buf.at[pl.ds(0, tail)])
                pltpu.sync_copy(buf.at[pl.ds(0, tail)],
                                tab_hbm.at[pl.ds(dst0 + o, tail)])


def kernel(x, tables):
    n_pts = x.shape[0]
    assert n_pts % (_NW * _C) == 0
    npt = n_pts // _NW
    xf = x.reshape(-1)
    flats = [t.reshape(-1) for t in tables]
    mesh = plsc.VectorSubcoreMesh(core_axis_name="c", subcore_axis_name="s")
    kfn = pl.kernel(
        functools.partial(_grid_body, npt, n_pts),
        out_type=jax.ShapeDtypeStruct((n_pts * 2 * _NL,), jnp.float32),
        mesh=mesh,
        scratch_types=[
            pltpu.VMEM((3 * npt,), jnp.float32),
            pltpu.VMEM((_NL * 2 * _L,), jnp.float32),
            pltpu.VMEM((_NL * 4 * _L,), jnp.int32),
            pltpu.VMEM((2, 1, _B2), jnp.int32),
            pltpu.VMEM((2 * _B,), jnp.float32),
            pltpu.VMEM((2, 1, _B2), jnp.float32),
            pltpu.VMEM((2 * _C * 32,), jnp.float32),
            pltpu.VMEM((_CHUNKW // 2,), jnp.float32),
            pltpu.VMEM((2 * (_SIZES[0] + _SIZES[1]),), jnp.float32),
            pltpu.HBM((_TOTAL_WORDS,), jnp.float32),
            pltpu.SemaphoreType.DMA((2,)),
        ],
        compiler_params=pltpu.CompilerParams(needs_layout_passes=False),
    )
    out = kfn(xf, *flats, jnp.asarray(_PF), jnp.asarray(_PI))
    return out.reshape(n_pts, 2 * _NL)
